# baseline (device time: 105667 ns/iter reference)
import jax
import jax.numpy as jnp
from jax import lax
from jax.experimental import pallas as pl
from jax.experimental.pallas import tpu as pltpu

N_DEV = 8
KC = 2
NB = 16


def kernel(x, w_mat, scale_x, scale_w):
    m_global, k_shard = x.shape
    k_global, n = w_mat.shape
    m_per = m_global // N_DEV
    bk = k_global // KC
    bn = n // NB
    slots_per_phase = N_DEV // KC

    def body(x_ref, w_ref, sx_ref, sw_ref, out_ref,
             xs_ref, comm_ref, a_ref, send_sems, recv_sems):
        c = pl.program_id(0)
        nb = pl.program_id(1)
        my = lax.axis_index("i")

        def peer_rdma(d, slot_dst, slot_sem):
            return pltpu.make_async_remote_copy(
                src_ref=xs_ref.at[pl.ds(d * m_per, m_per), :],
                dst_ref=comm_ref.at[slot_dst],
                send_sem=send_sems.at[slot_sem],
                recv_sem=recv_sems.at[slot_dst],
                device_id=(d,),
                device_id_type=pl.DeviceIdType.MESH,
            )

        @pl.when((c == 0) & (nb == 0))
        def _():
            xs_ref[...] = x_ref[...].astype(jnp.float8_e4m3fn)
            comm_ref[my] = xs_ref[pl.ds(my * m_per, m_per), :]
            for off in range(1, N_DEV):
                d = lax.rem(my + off, N_DEV)
                peer_rdma(d, my, d).start()

        @pl.when(nb == 0)
        def _():
            for j in range(slots_per_phase):
                s = c * slots_per_phase + j

                @pl.when(s != my)
                def _():
                    peer_rdma(my, s, s).wait_recv()

                a_ref[:, pl.ds(s * m_per, m_per)] = (
                    comm_ref[s].astype(jnp.bfloat16))

        wb = w_ref[...].astype(jnp.bfloat16)
        a = a_ref[:, pl.ds(c * bk, bk)]
        partial = jnp.dot(a, wb, preferred_element_type=jnp.float32)

        @pl.when(c == 0)
        def _():
            out_ref[:, pl.ds(nb * bn, bn)] = partial

        @pl.when(c == KC - 1)
        def _():
            s = sx_ref[0] * sw_ref[0]
            prev = out_ref[:, pl.ds(nb * bn, bn)]
            out_ref[:, pl.ds(nb * bn, bn)] = jnp.maximum(
                (prev + partial) * s, 0.0)

        @pl.when((c == KC - 1) & (nb == NB - 1))
        def _():
            for off in range(1, N_DEV):
                d = lax.rem(my + off, N_DEV)
                peer_rdma(d, my, d).wait_send()

    return pl.pallas_call(
        body,
        grid=(KC, NB),
        in_specs=[
            pl.BlockSpec((m_global, k_shard), lambda c, nb: (0, 0),
                         memory_space=pltpu.VMEM),
            pl.BlockSpec((bk, bn), lambda c, nb: (c, nb),
                         memory_space=pltpu.VMEM),
            pl.BlockSpec(memory_space=pltpu.SMEM),
            pl.BlockSpec(memory_space=pltpu.SMEM),
        ],
        out_specs=pl.BlockSpec((m_per, n), lambda c, nb: (0, 0),
                               memory_space=pltpu.VMEM),
        out_shape=jax.ShapeDtypeStruct((m_per, n), jnp.float32),
        scratch_shapes=[
            pltpu.VMEM((m_global, k_shard), jnp.float8_e4m3fn),
            pltpu.VMEM((N_DEV, m_per, k_shard), jnp.float8_e4m3fn),
            pltpu.VMEM((m_per, k_global), jnp.bfloat16),
            pltpu.SemaphoreType.DMA((N_DEV,)),
            pltpu.SemaphoreType.DMA((N_DEV,)),
        ],
        compiler_params=pltpu.CompilerParams(
            dimension_semantics=("arbitrary", "arbitrary"),
            vmem_limit_bytes=60 * 1024 * 1024,
        ),
    )(x, w_mat, scale_x, scale_w)


# device time: 95491 ns/iter; 1.1066x vs baseline; 1.1066x over previous
import jax
import jax.numpy as jnp
from jax import lax
from jax.experimental import pallas as pl
from jax.experimental.pallas import tpu as pltpu

N_DEV = 8
KC = 2
NB = 16
NBUF = 5
TOT = KC * NB
SPP = N_DEV // KC


def kernel(x, w_mat, scale_x, scale_w):
    m_global, k_shard = x.shape
    k_global, n = w_mat.shape
    m_per = m_global // N_DEV
    bk = k_global // KC
    bn = n // NB
    bks = k_global // N_DEV

    def body(x_ref, w_ref, sx_ref, sw_ref, out_ref,
             xs_ref, comm_ref, a_ref, wbuf_ref,
             send_sems, recv_sems, wsems):
        c = pl.program_id(0)
        nb = pl.program_id(1)
        t = c * NB + nb
        my = lax.axis_index("i")

        def peer_rdma(d, slot_dst, slot_sem):
            return pltpu.make_async_remote_copy(
                src_ref=xs_ref.at[pl.ds(d * m_per, m_per), :],
                dst_ref=comm_ref.at[slot_dst],
                send_sem=send_sems.at[slot_sem],
                recv_sem=recv_sems.at[slot_dst],
                device_id=(d,),
                device_id_type=pl.DeviceIdType.MESH,
            )

        def wsub(b, jj):
            cb = b // NB
            nbb = lax.rem(b, NB)
            slot = lax.rem(b, NBUF)
            sr = lax.rem(my + cb * SPP + jj, N_DEV)
            return pltpu.make_async_copy(
                w_ref.at[pl.ds(sr * bks, bks), pl.ds(nbb * bn, bn)],
                wbuf_ref.at[slot, pl.ds(jj * bks, bks), :],
                wsems.at[slot, jj],
            )

        def wstart(b):
            for jj in range(SPP):
                wsub(b, jj).start()

        @pl.when(t == 0)
        def _():
            for b in range(NBUF - 1):
                wstart(b)
            xs_ref[...] = x_ref[...].astype(jnp.float8_e4m3fn)
            comm_ref[my] = xs_ref[pl.ds(my * m_per, m_per), :]
            for off in range(1, N_DEV):
                d = lax.rem(my + off, N_DEV)
                peer_rdma(d, my, d).start()

        @pl.when(nb == 0)
        def _():
            for jj in range(SPP):
                j = c * SPP + jj
                s = lax.rem(my + j, N_DEV)

                @pl.when(j != 0)
                def _():
                    peer_rdma(my, s, s).wait_recv()

                a_ref[:, pl.ds(j * m_per, m_per)] = (
                    comm_ref[s].astype(jnp.bfloat16))

        for jj in range(SPP):
            wsub(t, jj).wait()
        wb = wbuf_ref[lax.rem(t, NBUF)].astype(jnp.bfloat16)
        a = a_ref[:, pl.ds(c * bk, bk)]
        partial = jnp.dot(a, wb, preferred_element_type=jnp.float32)

        @pl.when(c == 0)
        def _():
            out_ref[:, pl.ds(nb * bn, bn)] = partial

        @pl.when(c == KC - 1)
        def _():
            s = sx_ref[0] * sw_ref[0]
            prev = out_ref[:, pl.ds(nb * bn, bn)]
            out_ref[:, pl.ds(nb * bn, bn)] = jnp.maximum(
                (prev + partial) * s, 0.0)

        @pl.when(t + NBUF - 1 < TOT)
        def _():
            wstart(t + NBUF - 1)

        @pl.when(t == TOT - 1)
        def _():
            for off in range(1, N_DEV):
                d = lax.rem(my + off, N_DEV)
                peer_rdma(d, my, d).wait_send()

    return pl.pallas_call(
        body,
        grid=(KC, NB),
        in_specs=[
            pl.BlockSpec((m_global, k_shard), lambda c, nb: (0, 0),
                         memory_space=pltpu.VMEM),
            pl.BlockSpec(memory_space=pl.ANY),
            pl.BlockSpec(memory_space=pltpu.SMEM),
            pl.BlockSpec(memory_space=pltpu.SMEM),
        ],
        out_specs=pl.BlockSpec((m_per, n), lambda c, nb: (0, 0),
                               memory_space=pltpu.VMEM),
        out_shape=jax.ShapeDtypeStruct((m_per, n), jnp.float32),
        scratch_shapes=[
            pltpu.VMEM((m_global, k_shard), jnp.float8_e4m3fn),
            pltpu.VMEM((N_DEV, m_per, k_shard), jnp.float8_e4m3fn),
            pltpu.VMEM((m_per, k_global), jnp.bfloat16),
            pltpu.VMEM((NBUF, bk, bn), jnp.float32),
            pltpu.SemaphoreType.DMA((N_DEV,)),
            pltpu.SemaphoreType.DMA((N_DEV,)),
            pltpu.SemaphoreType.DMA((NBUF, SPP)),
        ],
        compiler_params=pltpu.CompilerParams(
            dimension_semantics=("arbitrary", "arbitrary"),
            vmem_limit_bytes=60 * 1024 * 1024,
        ),
    )(x, w_mat, scale_x, scale_w)


# device time: 90989 ns/iter; 1.1613x vs baseline; 1.0495x over previous
import jax
import jax.numpy as jnp
from jax import lax
from jax.experimental import pallas as pl
from jax.experimental.pallas import tpu as pltpu

N_DEV = 8
T_A = 16
NB_B = 32
TOT = T_A + NB_B
BN_A = 512
BN_B = 256
NBUFB = 5
NREM = N_DEV - 1


def kernel(x, w_mat, scale_x, scale_w):
    m_global, k_shard = x.shape
    k_global, n = w_mat.shape
    m_per = m_global // N_DEV
    bks = k_global // N_DEV

    def body(x_ref, w_ref, sx_ref, sw_ref, out_ref,
             xs_ref, comm_ref, a_ref, wbufa_ref, wbufb_ref,
             send_sems, recv_sems, wasems, wbsems):
        t = pl.program_id(0)
        my = lax.axis_index("i")

        def peer_rdma(d, slot_dst, slot_sem):
            return pltpu.make_async_remote_copy(
                src_ref=xs_ref.at[pl.ds(d * m_per, m_per), :],
                dst_ref=comm_ref.at[slot_dst],
                send_sem=send_sems.at[slot_sem],
                recv_sem=recv_sems.at[slot_dst],
                device_id=(d,),
                device_id_type=pl.DeviceIdType.MESH,
            )

        def wcopy_a(b):
            return pltpu.make_async_copy(
                w_ref.at[pl.ds(my * bks, bks), pl.ds(b * BN_A, BN_A)],
                wbufa_ref.at[lax.rem(b, 2)],
                wasems.at[lax.rem(b, 2)],
            )

        def wsub_b(b, jj):
            sr = lax.rem(my + 1 + jj, N_DEV)
            slot = lax.rem(b, NBUFB)
            return pltpu.make_async_copy(
                w_ref.at[pl.ds(sr * bks, bks), pl.ds(b * BN_B, BN_B)],
                wbufb_ref.at[slot, pl.ds(jj * bks, bks), :],
                wbsems.at[slot, jj],
            )

        def wstart_b(b):
            for jj in range(NREM):
                wsub_b(b, jj).start()

        @pl.when(t == 0)
        def _():
            wcopy_a(0).start()
            wcopy_a(1).start()
            for b in range(NBUFB - 1):
                wstart_b(b)
            xs_ref[...] = x_ref[...].astype(jnp.float8_e4m3fn)
            comm_ref[my] = xs_ref[pl.ds(my * m_per, m_per), :]
            for off in range(1, N_DEV):
                d = lax.rem(my + off, N_DEV)
                peer_rdma(d, my, d).start()
            a_ref[:, pl.ds(0, bks)] = comm_ref[my].astype(jnp.bfloat16)

        @pl.when(t < T_A)
        def _():
            wcopy_a(t).wait()
            wba = wbufa_ref[lax.rem(t, 2)].astype(jnp.bfloat16)
            partial = jnp.dot(a_ref[:, pl.ds(0, bks)], wba,
                              preferred_element_type=jnp.float32)
            out_ref[:, pl.ds(t * BN_A, BN_A)] = partial

            @pl.when(t + 2 < T_A)
            def _():
                wcopy_a(t + 2).start()

        @pl.when(t == T_A)
        def _():
            for jj in range(NREM):
                s = lax.rem(my + 1 + jj, N_DEV)
                peer_rdma(my, s, s).wait_recv()
                a_ref[:, pl.ds((1 + jj) * bks, bks)] = (
                    comm_ref[s].astype(jnp.bfloat16))

        @pl.when(t >= T_A)
        def _():
            i = t - T_A
            for jj in range(NREM):
                wsub_b(i, jj).wait()
            wbb = wbufb_ref[lax.rem(i, NBUFB)].astype(jnp.bfloat16)
            partial = jnp.dot(a_ref[:, pl.ds(bks, NREM * bks)], wbb,
                              preferred_element_type=jnp.float32)
            s = sx_ref[0] * sw_ref[0]
            prev = out_ref[:, pl.ds(i * BN_B, BN_B)]
            out_ref[:, pl.ds(i * BN_B, BN_B)] = jnp.maximum(
                (prev + partial) * s, 0.0)

            @pl.when(i + NBUFB - 1 < NB_B)
            def _():
                wstart_b(i + NBUFB - 1)

        @pl.when(t == TOT - 1)
        def _():
            for off in range(1, N_DEV):
                d = lax.rem(my + off, N_DEV)
                peer_rdma(d, my, d).wait_send()

    return pl.pallas_call(
        body,
        grid=(TOT,),
        in_specs=[
            pl.BlockSpec((m_global, k_shard), lambda t: (0, 0),
                         memory_space=pltpu.VMEM),
            pl.BlockSpec(memory_space=pl.ANY),
            pl.BlockSpec(memory_space=pltpu.SMEM),
            pl.BlockSpec(memory_space=pltpu.SMEM),
        ],
        out_specs=pl.BlockSpec((m_per, n), lambda t: (0, 0),
                               memory_space=pltpu.VMEM),
        out_shape=jax.ShapeDtypeStruct((m_per, n), jnp.float32),
        scratch_shapes=[
            pltpu.VMEM((m_global, k_shard), jnp.float8_e4m3fn),
            pltpu.VMEM((N_DEV, m_per, k_shard), jnp.float8_e4m3fn),
            pltpu.VMEM((m_per, k_global), jnp.bfloat16),
            pltpu.VMEM((2, bks, BN_A), jnp.float32),
            pltpu.VMEM((NBUFB, NREM * bks, BN_B), jnp.float32),
            pltpu.SemaphoreType.DMA((N_DEV,)),
            pltpu.SemaphoreType.DMA((N_DEV,)),
            pltpu.SemaphoreType.DMA((2,)),
            pltpu.SemaphoreType.DMA((NBUFB, NREM)),
        ],
        compiler_params=pltpu.CompilerParams(
            dimension_semantics=("arbitrary",),
            vmem_limit_bytes=60 * 1024 * 1024,
        ),
    )(x, w_mat, scale_x, scale_w)


# device time: 87850 ns/iter; 1.2028x vs baseline; 1.0357x over previous
import jax
import jax.numpy as jnp
from jax import lax
from jax.experimental import pallas as pl
from jax.experimental.pallas import tpu as pltpu

N_DEV = 8
T_A = 16
NB_B = 16
TOT = T_A + NB_B
BN_A = 512
BN_B = 512
NBUFB = 3
NREM = N_DEV - 1


def kernel(x, w_mat, scale_x, scale_w):
    m_global, k_shard = x.shape
    k_global, n = w_mat.shape
    m_per = m_global // N_DEV
    bks = k_global // N_DEV

    def body(x_ref, w_ref, sx_ref, sw_ref, out_ref,
             xs_ref, comm_ref, a_ref, wbufa_ref, wbufb_ref,
             send_sems, recv_sems, wasems, wbsems):
        t = pl.program_id(0)
        my = lax.axis_index("i")

        def peer_rdma(d, slot_dst, slot_sem):
            return pltpu.make_async_remote_copy(
                src_ref=xs_ref.at[pl.ds(d * m_per, m_per), :],
                dst_ref=comm_ref.at[slot_dst],
                send_sem=send_sems.at[slot_sem],
                recv_sem=recv_sems.at[slot_dst],
                device_id=(d,),
                device_id_type=pl.DeviceIdType.MESH,
            )

        def wcopy_a(b):
            return pltpu.make_async_copy(
                w_ref.at[pl.ds(my * bks, bks), pl.ds(b * BN_A, BN_A)],
                wbufa_ref.at[lax.rem(b, 2)],
                wasems.at[lax.rem(b, 2)],
            )

        def wsub_b(b, jj):
            sr = lax.rem(my + 1 + jj, N_DEV)
            slot = lax.rem(b, NBUFB)
            return pltpu.make_async_copy(
                w_ref.at[pl.ds(sr * bks, bks), pl.ds(b * BN_B, BN_B)],
                wbufb_ref.at[slot, pl.ds(jj * bks, bks), :],
                wbsems.at[slot, jj],
            )

        def wstart_b(b):
            for jj in range(NREM):
                wsub_b(b, jj).start()

        @pl.when(t == 0)
        def _():
            wcopy_a(0).start()
            wcopy_a(1).start()
            for b in range(NBUFB - 1):
                wstart_b(b)
            xs_ref[...] = x_ref[...].astype(jnp.float8_e4m3fn)
            comm_ref[my] = xs_ref[pl.ds(my * m_per, m_per), :]
            for off in range(1, N_DEV):
                d = lax.rem(my + off, N_DEV)
                peer_rdma(d, my, d).start()
            a_ref[:, pl.ds(0, bks)] = comm_ref[my].astype(jnp.bfloat16)

        @pl.when(t < T_A)
        def _():
            wcopy_a(t).wait()
            wba = wbufa_ref[lax.rem(t, 2)].astype(jnp.bfloat16)
            partial = jnp.dot(a_ref[:, pl.ds(0, bks)], wba,
                              preferred_element_type=jnp.float32)
            out_ref[:, pl.ds(t * BN_A, BN_A)] = partial

            @pl.when(t + 2 < T_A)
            def _():
                wcopy_a(t + 2).start()

        @pl.when(t == T_A)
        def _():
            for jj in range(NREM):
                s = lax.rem(my + 1 + jj, N_DEV)
                peer_rdma(my, s, s).wait_recv()
                a_ref[:, pl.ds((1 + jj) * bks, bks)] = (
                    comm_ref[s].astype(jnp.bfloat16))

        @pl.when(t >= T_A)
        def _():
            i = t - T_A
            for jj in range(NREM):
                wsub_b(i, jj).wait()
            wbb = wbufb_ref[lax.rem(i, NBUFB)].astype(jnp.bfloat16)
            partial = jnp.dot(a_ref[:, pl.ds(bks, NREM * bks)], wbb,
                              preferred_element_type=jnp.float32)
            s = sx_ref[0] * sw_ref[0]
            prev = out_ref[:, pl.ds(i * BN_B, BN_B)]
            out_ref[:, pl.ds(i * BN_B, BN_B)] = jnp.maximum(
                (prev + partial) * s, 0.0)

            @pl.when(i + NBUFB - 1 < NB_B)
            def _():
                wstart_b(i + NBUFB - 1)

        @pl.when(t == TOT - 1)
        def _():
            for off in range(1, N_DEV):
                d = lax.rem(my + off, N_DEV)
                peer_rdma(d, my, d).wait_send()

    return pl.pallas_call(
        body,
        grid=(TOT,),
        in_specs=[
            pl.BlockSpec((m_global, k_shard), lambda t: (0, 0),
                         memory_space=pltpu.VMEM),
            pl.BlockSpec(memory_space=pl.ANY),
            pl.BlockSpec(memory_space=pltpu.SMEM),
            pl.BlockSpec(memory_space=pltpu.SMEM),
        ],
        out_specs=pl.BlockSpec((m_per, n), lambda t: (0, 0),
                               memory_space=pltpu.VMEM),
        out_shape=jax.ShapeDtypeStruct((m_per, n), jnp.float32),
        scratch_shapes=[
            pltpu.VMEM((m_global, k_shard), jnp.float8_e4m3fn),
            pltpu.VMEM((N_DEV, m_per, k_shard), jnp.float8_e4m3fn),
            pltpu.VMEM((m_per, k_global), jnp.bfloat16),
            pltpu.VMEM((2, bks, BN_A), jnp.float32),
            pltpu.VMEM((NBUFB, NREM * bks, BN_B), jnp.float32),
            pltpu.SemaphoreType.DMA((N_DEV,)),
            pltpu.SemaphoreType.DMA((N_DEV,)),
            pltpu.SemaphoreType.DMA((2,)),
            pltpu.SemaphoreType.DMA((NBUFB, NREM)),
        ],
        compiler_params=pltpu.CompilerParams(
            dimension_semantics=("arbitrary",),
            vmem_limit_bytes=61 * 1024 * 1024,
        ),
    )(x, w_mat, scale_x, scale_w)
